# baseline (speedup 1.0000x reference)
import functools

import jax
import jax.numpy as jnp
from jax import lax
from jax.experimental import pallas as pl
from jax.experimental.pallas import tpu as pltpu
from jax.experimental.pallas import tpu_sc as plsc

BATCH = 16384
EMBED_DIM = 32
HIDDEN_DIM = 64

_NUM_CORES = 2
_NUM_SUBCORES = 16
_NW = _NUM_CORES * _NUM_SUBCORES
_B_PER_W = BATCH // _NW
_CHUNK = 128
_NCHUNK = _B_PER_W // _CHUNK


def _gather_body(uid_hbm, iid_hbm, uemb_hbm, iemb_hbm, out_u, out_i,
                 idx_v, rows_u, rows_i, sem):
    wid = lax.axis_index("s") * _NUM_CORES + lax.axis_index("c")
    base = wid * _B_PER_W
    stage = []
    for j in range(_NCHUNK):
        stage.append(pltpu.async_copy(
            uid_hbm.at[pl.ds(base + j * _CHUNK, _CHUNK)], idx_v.at[j], sem))
        stage.append(pltpu.async_copy(
            iid_hbm.at[pl.ds(base + j * _CHUNK, _CHUNK)],
            idx_v.at[_NCHUNK + j], sem))
    for c in stage:
        c.wait()
    copies = []
    for j in range(_NCHUNK):
        copies.append(pltpu.async_copy(
            uemb_hbm.at[idx_v.at[j]],
            rows_u.at[pl.ds(j * _CHUNK, _CHUNK)], sem))
    for j in range(_NCHUNK):
        copies.append(pltpu.async_copy(
            iemb_hbm.at[idx_v.at[_NCHUNK + j]],
            rows_i.at[pl.ds(j * _CHUNK, _CHUNK)], sem))
    for c in copies:
        c.wait()
    pltpu.sync_copy(rows_u, out_u.at[pl.ds(base, _B_PER_W)])
    pltpu.sync_copy(rows_i, out_i.at[pl.ds(base, _B_PER_W)])


@functools.cache
def _sc_gather():
    return pl.kernel(
        _gather_body,
        out_type=(
            jax.ShapeDtypeStruct((BATCH, EMBED_DIM), jnp.float32),
            jax.ShapeDtypeStruct((BATCH, EMBED_DIM), jnp.float32),
        ),
        mesh=plsc.VectorSubcoreMesh(core_axis_name="c", subcore_axis_name="s"),
        scratch_types=[
            pltpu.VMEM((2 * _NCHUNK, _CHUNK), jnp.int32),
            pltpu.VMEM((_B_PER_W, EMBED_DIM), jnp.float32),
            pltpu.VMEM((_B_PER_W, EMBED_DIM), jnp.float32),
            pltpu.SemaphoreType.DMA,
        ],
        compiler_params=pltpu.CompilerParams(use_tc_tiling_on_sc=False),
    )


_MLP_BLK = 2048


def _mlp_body(ue_ref, ie_ref, w1u_ref, w1i_ref, b1_ref, w2_ref, b2_ref,
              out_ref):
    h = jnp.dot(ue_ref[...], w1u_ref[...], preferred_element_type=jnp.float32)
    h = h + jnp.dot(ie_ref[...], w1i_ref[...],
                    preferred_element_type=jnp.float32)
    h = jnp.maximum(h + b1_ref[...], 0.0)
    y = jnp.sum(h * w2_ref[...], axis=1) + b2_ref[0, 0]
    out_ref[...] = jax.nn.sigmoid(y)


def _mlp_call(ue, ie, w1u, w1i, b1, w2, b2):
    grid = BATCH // _MLP_BLK
    return pl.pallas_call(
        _mlp_body,
        grid=(grid,),
        in_specs=[
            pl.BlockSpec((_MLP_BLK, EMBED_DIM), lambda i: (i, 0)),
            pl.BlockSpec((_MLP_BLK, EMBED_DIM), lambda i: (i, 0)),
            pl.BlockSpec((EMBED_DIM, HIDDEN_DIM), lambda i: (0, 0)),
            pl.BlockSpec((EMBED_DIM, HIDDEN_DIM), lambda i: (0, 0)),
            pl.BlockSpec((1, HIDDEN_DIM), lambda i: (0, 0)),
            pl.BlockSpec((1, HIDDEN_DIM), lambda i: (0, 0)),
            pl.BlockSpec((1, 1), lambda i: (0, 0)),
        ],
        out_specs=pl.BlockSpec((_MLP_BLK,), lambda i: (i,)),
        out_shape=jax.ShapeDtypeStruct((BATCH,), jnp.float32),
    )(ue, ie, w1u, w1i, b1, w2, b2)


def kernel(user_ids, item_ids, user_emb, item_emb, W1, b1, W2, b2):
    ue, ie = _sc_gather()(user_ids, item_ids, user_emb, item_emb)
    w1u = W1[:, :EMBED_DIM].T
    w1i = W1[:, EMBED_DIM:].T
    b1r = b1.reshape(1, HIDDEN_DIM)
    w2r = W2.reshape(1, HIDDEN_DIM)
    b2r = b2.reshape(1, 1)
    return _mlp_call(ue, ie, w1u, w1i, b1r, w2r, b2r)
